# baseline (speedup 1.0000x reference)
import functools

import jax
import jax.numpy as jnp
from jax import lax
from jax.experimental import pallas as pl
from jax.experimental.pallas import tpu as pltpu
from jax.experimental.pallas import tpu_sc as plsc

B, F, D = 4096, 26, 32
NC, NS = 2, 16
NW = NC * NS
RPW = B // NW
NL = RPW // 16
NCHUNK = 4
DC = D // NCHUNK


def _fm_body(st_hbm, et_hbm, w_hbm, out_hbm,
             slab_v, gath_v, eb0, eb1, eb2, eb3, out_v,
             gsem, s0, s1, s2, s3):
    wid = lax.axis_index("s") * NC + lax.axis_index("c")
    base = wid * RPW
    pltpu.sync_copy(st_hbm.at[:, pl.ds(base, RPW)], slab_v)

    ebufs = (eb0, eb1, eb2, eb3)
    sems = (s0, s1, s2, s3)
    chunks = [
        pltpu.async_copy(
            et_hbm.at[:, pl.ds(c * DC, DC), pl.ds(base, RPW)], ebufs[c], sems[c])
        for c in range(NCHUNK)
    ]
    gathers = [
        pltpu.async_copy(w_hbm.at[0].at[slab_v.at[f]], gath_v.at[f], gsem)
        for f in range(F)
    ]

    zero16 = jnp.zeros((16,), jnp.float32)
    for l in range(NL):
        out_v[pl.ds(l * 16, 16)] = zero16
    for c in range(NCHUNK):
        chunks[c].wait()
        ebuf = ebufs[c]
        for l in range(NL):
            def dbody(dd, acc2, ebuf=ebuf, l=l):
                s = zero16
                q = zero16
                for f in range(F):
                    v = ebuf[f, dd, pl.ds(l * 16, 16)]
                    s = s + v
                    q = q + v * v
                return acc2 + (s * s - q)

            acc2 = lax.fori_loop(0, DC, dbody, zero16)
            out_v[pl.ds(l * 16, 16)] = out_v[pl.ds(l * 16, 16)] + 0.5 * acc2

    for g in gathers:
        g.wait()

    for l in range(NL):
        fo = zero16
        for f in range(F):
            fo = fo + gath_v[f, pl.ds(l * 16, 16)]
        out_v[pl.ds(l * 16, 16)] = out_v[pl.ds(l * 16, 16)] + fo

    pltpu.sync_copy(out_v, out_hbm.at[pl.ds(base, RPW)])


@jax.jit
def kernel(sparse_inputs, embed_inputs, w):
    run = pl.kernel(
        _fm_body,
        out_type=jax.ShapeDtypeStruct((B,), jnp.float32),
        mesh=plsc.VectorSubcoreMesh(core_axis_name="c", subcore_axis_name="s"),
        scratch_types=[
            pltpu.VMEM((F, RPW), jnp.int32),
            pltpu.VMEM((F, RPW), jnp.float32),
            pltpu.VMEM((F, DC, RPW), jnp.float32),
            pltpu.VMEM((F, DC, RPW), jnp.float32),
            pltpu.VMEM((F, DC, RPW), jnp.float32),
            pltpu.VMEM((F, DC, RPW), jnp.float32),
            pltpu.VMEM((RPW,), jnp.float32),
            pltpu.SemaphoreType.DMA,
            pltpu.SemaphoreType.DMA,
            pltpu.SemaphoreType.DMA,
            pltpu.SemaphoreType.DMA,
            pltpu.SemaphoreType.DMA,
        ],
        compiler_params=pltpu.CompilerParams(needs_layout_passes=False),
    )
    st = sparse_inputs.T
    et = embed_inputs.transpose(1, 2, 0)
    wt = w.T
    return run(st, et, wt).reshape(B, 1)


# speedup vs baseline: 1.4158x; 1.4158x over previous
import functools

import jax
import jax.numpy as jnp
from jax import lax
from jax.experimental import pallas as pl
from jax.experimental.pallas import tpu as pltpu
from jax.experimental.pallas import tpu_sc as plsc

B, F, D = 4096, 26, 32
NC, NS = 2, 16
NW = NC * NS
RPW = B // NW
NL = RPW // 16
NCHUNK = 4
DC = D // NCHUNK


def _fm_body(st_hbm, et_hbm, w_hbm, out_hbm,
             slab_v, gath_v, emb_v, out_v, gsem, s0, s1, s2, s3):
    wid = lax.axis_index("s") * NC + lax.axis_index("c")
    base = wid * RPW
    pltpu.sync_copy(st_hbm.at[:, pl.ds(base, RPW)], slab_v)

    sems = (s0, s1, s2, s3)
    chunks = [
        pltpu.async_copy(et_hbm.at[:, pl.ds(c * DC, DC), pl.ds(base, RPW)],
                         emb_v.at[:, pl.ds(c * DC, DC), :], sems[c])
        for c in range(NCHUNK)
    ]
    gathers = [
        pltpu.async_copy(w_hbm.at[0].at[slab_v.at[f]], gath_v.at[f], gsem)
        for f in range(F)
    ]

    zero16 = jnp.zeros((16,), jnp.float32)

    def dbody(dd, accs):
        for c in range(NCHUNK):
            @pl.when(dd == c * DC)
            def _(c=c):
                chunks[c].wait()
        new = []
        for l in range(NL):
            s = zero16
            q = zero16
            for f in range(F):
                v = emb_v[f, dd, pl.ds(l * 16, 16)]
                s = s + v
                q = q + v * v
            new.append(accs[l] + (s * s - q))
        return tuple(new)

    accs = lax.fori_loop(0, D, dbody, (zero16,) * NL)

    for g in gathers:
        g.wait()

    for l in range(NL):
        fo = zero16
        for f in range(F):
            fo = fo + gath_v[f, pl.ds(l * 16, 16)]
        out_v[pl.ds(l * 16, 16)] = fo + 0.5 * accs[l]

    pltpu.sync_copy(out_v, out_hbm.at[pl.ds(base, RPW)])


@jax.jit
def kernel(sparse_inputs, embed_inputs, w):
    run = pl.kernel(
        _fm_body,
        out_type=jax.ShapeDtypeStruct((B,), jnp.float32),
        mesh=plsc.VectorSubcoreMesh(core_axis_name="c", subcore_axis_name="s"),
        scratch_types=[
            pltpu.VMEM((F, RPW), jnp.int32),
            pltpu.VMEM((F, RPW), jnp.float32),
            pltpu.VMEM((F, D, RPW), jnp.float32),
            pltpu.VMEM((RPW,), jnp.float32),
            pltpu.SemaphoreType.DMA,
            pltpu.SemaphoreType.DMA,
            pltpu.SemaphoreType.DMA,
            pltpu.SemaphoreType.DMA,
            pltpu.SemaphoreType.DMA,
        ],
        compiler_params=pltpu.CompilerParams(needs_layout_passes=False),
    )
    st = sparse_inputs.T
    et = embed_inputs.transpose(1, 2, 0)
    wt = w.T
    return run(st, et, wt).reshape(B, 1)
